# baseline (speedup 1.0000x reference)
import functools

import jax
import jax.numpy as jnp
from jax import lax
from jax.experimental import pallas as pl
from jax.experimental.pallas import tpu as pltpu
from jax.experimental.pallas import tpu_sc as plsc

_BS = 2048
_SC_B = 1
_NW = 32
_CHUNK = 32768


def _tc_body(x_ref, p_ref, o_ref):
    x = x_ref[0]
    p = p_ref[...]
    o_ref[0] = x + jnp.where(x != 0.0, p, 0.0)


def _tc_part(x, pos_table):
    B, L, D = x.shape
    return pl.pallas_call(
        _tc_body,
        grid=(L // _BS, B),
        in_specs=[
            pl.BlockSpec((1, _BS, D), lambda s, b: (b, s, 0)),
            pl.BlockSpec((_BS, D), lambda s, b: (s, 0)),
        ],
        out_specs=pl.BlockSpec((1, _BS, D), lambda s, b: (b, s, 0)),
        out_shape=jax.ShapeDtypeStruct((B, L, D), x.dtype),
    )(x, pos_table)


def _sc_part(x_flat, p_flat):
    total = x_flat.shape[0]
    table_n = p_flat.shape[0]
    per_w = total // _NW
    n_chunks = per_w // _CHUNK

    @functools.partial(
        pl.kernel,
        out_type=jax.ShapeDtypeStruct((total,), jnp.float32),
        mesh=plsc.VectorSubcoreMesh(core_axis_name="c", subcore_axis_name="s"),
        scratch_types=[
            pltpu.VMEM((_CHUNK,), jnp.float32),
            pltpu.VMEM((_CHUNK,), jnp.float32),
        ],
    )
    def sc_kernel(x_hbm, p_hbm, o_hbm, xv, pv):
        wid = lax.axis_index("s") * 2 + lax.axis_index("c")
        wbase = wid * per_w
        for k in range(n_chunks):
            base = pl.multiple_of(wbase + k * _CHUNK, 8)
            toff = pl.multiple_of(lax.bitwise_and(base, table_n - 1), 8)
            pltpu.sync_copy(x_hbm.at[pl.ds(base, _CHUNK)], xv)
            pltpu.sync_copy(p_hbm.at[pl.ds(toff, _CHUNK)], pv)

            @plsc.parallel_loop(0, _CHUNK, step=16, unroll=8)
            def _(i):
                v = xv[pl.ds(i, 16)]
                t = pv[pl.ds(i, 16)]
                xv[pl.ds(i, 16)] = v + jnp.where(v != 0.0, t, jnp.zeros_like(t))

            pltpu.sync_copy(xv, o_hbm.at[pl.ds(base, _CHUNK)])

    return sc_kernel(x_flat, p_flat)


def kernel(inputs, pos_table):
    B, L, D = inputs.shape
    sc_out = _sc_part(inputs[:_SC_B].reshape(-1), pos_table.reshape(-1))
    tc_out = _tc_part(inputs[_SC_B:], pos_table)
    return jnp.concatenate([sc_out.reshape(_SC_B, L, D), tc_out], axis=0)


# speedup vs baseline: 1.0452x; 1.0452x over previous
import functools

import jax
import jax.numpy as jnp
from jax import lax
from jax.experimental import pallas as pl
from jax.experimental.pallas import tpu as pltpu
from jax.experimental.pallas import tpu_sc as plsc

_BS = 2048
_SC_B = 1
_NW = 32
_CHUNK = 32768


def _tc_body(x_ref, p_ref, o_ref):
    x = x_ref[0]
    p = p_ref[...]
    o_ref[0] = x + jnp.where(x != 0.0, p, 0.0)


def _tc_part(x, pos_table):
    B, L, D = x.shape
    return pl.pallas_call(
        _tc_body,
        grid=(L // _BS, B - _SC_B),
        in_specs=[
            pl.BlockSpec((1, _BS, D), lambda s, b: (b + _SC_B, s, 0)),
            pl.BlockSpec((_BS, D), lambda s, b: (s, 0)),
        ],
        out_specs=pl.BlockSpec((1, _BS, D), lambda s, b: (b, s, 0)),
        out_shape=jax.ShapeDtypeStruct((B - _SC_B, L, D), x.dtype),
    )(x, pos_table)


def _sc_part(x_flat, p_flat):
    table_n = p_flat.shape[0]
    total = _SC_B * table_n
    per_w = total // _NW
    n_chunks = per_w // _CHUNK

    @functools.partial(
        pl.kernel,
        out_type=jax.ShapeDtypeStruct((total,), jnp.float32),
        mesh=plsc.VectorSubcoreMesh(core_axis_name="c", subcore_axis_name="s"),
        scratch_types=[
            pltpu.VMEM((_CHUNK,), jnp.float32),
            pltpu.VMEM((_CHUNK,), jnp.float32),
        ],
    )
    def sc_kernel(x_hbm, p_hbm, o_hbm, xv, pv):
        wid = lax.axis_index("s") * 2 + lax.axis_index("c")
        wbase = wid * per_w
        for k in range(n_chunks):
            base = pl.multiple_of(wbase + k * _CHUNK, 8)
            toff = pl.multiple_of(lax.bitwise_and(base, table_n - 1), 8)
            pltpu.sync_copy(x_hbm.at[pl.ds(base, _CHUNK)], xv)
            pltpu.sync_copy(p_hbm.at[pl.ds(toff, _CHUNK)], pv)

            @plsc.parallel_loop(0, _CHUNK, step=16, unroll=8)
            def _(i):
                v = xv[pl.ds(i, 16)]
                t = pv[pl.ds(i, 16)]
                xv[pl.ds(i, 16)] = v + jnp.where(v != 0.0, t, jnp.zeros_like(t))

            pltpu.sync_copy(xv, o_hbm.at[pl.ds(base, _CHUNK)])

    return sc_kernel(x_flat, p_flat)


def kernel(inputs, pos_table):
    B, L, D = inputs.shape
    sc_out = _sc_part(inputs.reshape(-1), pos_table.reshape(-1))
    tc_out = _tc_part(inputs, pos_table)
    return jnp.concatenate([sc_out.reshape(_SC_B, L, D), tc_out], axis=0)


# speedup vs baseline: 4.3723x; 4.1833x over previous
import jax
import jax.numpy as jnp
from jax.experimental import pallas as pl

_BS = 2048


def _body(x_ref, p_ref, o_ref):
    x = x_ref[0]
    p = p_ref[...]
    o_ref[0] = x + jnp.where(x != 0.0, p, 0.0)


def kernel(inputs, pos_table):
    B, L, D = inputs.shape
    return pl.pallas_call(
        _body,
        grid=(L // _BS, B),
        in_specs=[
            pl.BlockSpec((1, _BS, D), lambda s, b: (b, s, 0)),
            pl.BlockSpec((_BS, D), lambda s, b: (s, 0)),
        ],
        out_specs=pl.BlockSpec((1, _BS, D), lambda s, b: (b, s, 0)),
        out_shape=jax.ShapeDtypeStruct((B, L, D), inputs.dtype),
    )(inputs, pos_table)


# speedup vs baseline: 4.3771x; 1.0011x over previous
import jax
import jax.numpy as jnp
from jax.experimental import pallas as pl

_BS = 1024


def _body(x_ref, p_ref, o_ref):
    x = x_ref[...]
    p = p_ref[...]
    o_ref[...] = x + jnp.where(x != 0.0, p, 0.0)


def kernel(inputs, pos_table):
    B, L, D = inputs.shape
    return pl.pallas_call(
        _body,
        grid=(L // _BS, B // 2),
        in_specs=[
            pl.BlockSpec((2, _BS, D), lambda s, b: (b, s, 0)),
            pl.BlockSpec((_BS, D), lambda s, b: (s, 0)),
        ],
        out_specs=pl.BlockSpec((2, _BS, D), lambda s, b: (b, s, 0)),
        out_shape=jax.ShapeDtypeStruct((B, L, D), inputs.dtype),
    )(inputs, pos_table)
